# baseline (speedup 1.0000x reference)
import functools

import jax
import jax.numpy as jnp
from jax import lax
from jax.experimental import pallas as pl
from jax.experimental.pallas import tpu as pltpu
from jax.experimental.pallas import tpu_sc as plsc

_MU = 3.5
_LANES = 16
_NCHUNK = 4


@jax.jit
def kernel(user, item, user_biases, item_biases):
    batch = user.shape[0]
    info = plsc.get_sparse_core_info()
    num_subcores = info.num_subcores
    num_cores = 1
    num_workers = num_cores * num_subcores
    b_per_w = batch // num_workers
    chunk = b_per_w // _NCHUNK

    mesh = plsc.VectorSubcoreMesh(core_axis_name="c", subcore_axis_name="s",
                                  num_cores=num_cores)

    idx_t = pltpu.VMEM((chunk,), jnp.int32)
    val_t = pltpu.VMEM((chunk,), jnp.float32)

    @functools.partial(
        pl.kernel,
        mesh=mesh,
        out_type=jax.ShapeDtypeStruct((batch,), jnp.float32),
        scratch_types=(
            [idx_t] * (2 * _NCHUNK)
            + [val_t] * (3 * _NCHUNK)
            + [pltpu.SemaphoreType.DMA] * (2 * _NCHUNK)
        ),
    )
    def sc_kernel(user_hbm, item_hbm, ub_hbm, ib_hbm, out_hbm, *scratch):
        uidx = scratch[0:_NCHUNK]
        iidx = scratch[_NCHUNK:2 * _NCHUNK]
        bu = scratch[2 * _NCHUNK:3 * _NCHUNK]
        bi = scratch[3 * _NCHUNK:4 * _NCHUNK]
        out = scratch[4 * _NCHUNK:5 * _NCHUNK]
        sem_u = scratch[5 * _NCHUNK:6 * _NCHUNK]
        sem_i = scratch[6 * _NCHUNK:7 * _NCHUNK]

        wid = lax.axis_index("s") * num_cores + lax.axis_index("c")
        base = wid * b_per_w
        sls = [pl.ds(base + c * chunk, chunk) for c in range(_NCHUNK)]

        cp_u = [pltpu.async_copy(user_hbm.at[sls[c]], uidx[c], sem_u[c])
                for c in range(_NCHUNK)]
        cp_i = [pltpu.async_copy(item_hbm.at[sls[c]], iidx[c], sem_i[c])
                for c in range(_NCHUNK)]
        g_u, g_i = [], []
        for c in range(_NCHUNK):
            cp_u[c].wait()
            g_u.append(pltpu.async_copy(ub_hbm.at[uidx[c]], bu[c], sem_u[c]))
            cp_i[c].wait()
            g_i.append(pltpu.async_copy(ib_hbm.at[iidx[c]], bi[c], sem_i[c]))
        wb = []
        for c in range(_NCHUNK):
            g_u[c].wait()
            g_i[c].wait()
            for i in range(chunk // _LANES):
                v = pl.ds(i * _LANES, _LANES)
                out[c][v] = bu[c][v] + bi[c][v] + _MU
            wb.append(pltpu.async_copy(out[c], out_hbm.at[sls[c]], sem_u[c]))
        for c in range(_NCHUNK):
            wb[c].wait()

    return sc_kernel(
        user.astype(jnp.int32),
        item.astype(jnp.int32),
        user_biases.reshape(-1),
        item_biases.reshape(-1),
    )


# speedup vs baseline: 1.0009x; 1.0009x over previous
import functools

import jax
import jax.numpy as jnp
from jax import lax
from jax.experimental import pallas as pl
from jax.experimental.pallas import tpu as pltpu
from jax.experimental.pallas import tpu_sc as plsc

_MU = 3.5
_LANES = 16
_NCHUNK = 2


@jax.jit
def kernel(user, item, user_biases, item_biases):
    batch = user.shape[0]
    info = plsc.get_sparse_core_info()
    num_subcores = info.num_subcores
    num_cores = 1
    num_workers = num_cores * num_subcores
    b_per_w = batch // num_workers
    chunk = b_per_w // _NCHUNK

    mesh = plsc.VectorSubcoreMesh(core_axis_name="c", subcore_axis_name="s",
                                  num_cores=num_cores)

    idx_t = pltpu.VMEM((chunk,), jnp.int32)
    val_t = pltpu.VMEM((chunk,), jnp.float32)

    @functools.partial(
        pl.kernel,
        mesh=mesh,
        out_type=jax.ShapeDtypeStruct((batch,), jnp.float32),
        scratch_types=(
            [idx_t] * (2 * _NCHUNK)
            + [val_t] * (3 * _NCHUNK)
            + [pltpu.SemaphoreType.DMA] * (2 * _NCHUNK)
        ),
    )
    def sc_kernel(user_hbm, item_hbm, ub_hbm, ib_hbm, out_hbm, *scratch):
        uidx = scratch[0:_NCHUNK]
        iidx = scratch[_NCHUNK:2 * _NCHUNK]
        bu = scratch[2 * _NCHUNK:3 * _NCHUNK]
        bi = scratch[3 * _NCHUNK:4 * _NCHUNK]
        out = scratch[4 * _NCHUNK:5 * _NCHUNK]
        sem_u = scratch[5 * _NCHUNK:6 * _NCHUNK]
        sem_i = scratch[6 * _NCHUNK:7 * _NCHUNK]

        wid = lax.axis_index("s") * num_cores + lax.axis_index("c")
        base = wid * b_per_w
        sls = [pl.ds(base + c * chunk, chunk) for c in range(_NCHUNK)]

        cp_u = [pltpu.async_copy(user_hbm.at[sls[c]], uidx[c], sem_u[c])
                for c in range(_NCHUNK)]
        cp_i = [pltpu.async_copy(item_hbm.at[sls[c]], iidx[c], sem_i[c])
                for c in range(_NCHUNK)]
        g_u, g_i = [], []
        for c in range(_NCHUNK):
            cp_u[c].wait()
            g_u.append(pltpu.async_copy(ub_hbm.at[uidx[c]], bu[c], sem_u[c]))
            cp_i[c].wait()
            g_i.append(pltpu.async_copy(ib_hbm.at[iidx[c]], bi[c], sem_i[c]))
        wb = []
        for c in range(_NCHUNK):
            g_u[c].wait()
            g_i[c].wait()
            for i in range(chunk // _LANES):
                v = pl.ds(i * _LANES, _LANES)
                out[c][v] = (_MU + bu[c][v]) + bi[c][v]
            wb.append(pltpu.async_copy(out[c], out_hbm.at[sls[c]], sem_u[c]))
        for c in range(_NCHUNK):
            wb[c].wait()

    return sc_kernel(
        user.astype(jnp.int32),
        item.astype(jnp.int32),
        user_biases.reshape(-1),
        item_biases.reshape(-1),
    )
